# baseline (speedup 1.0000x reference)
import functools

import jax
import jax.numpy as jnp
from jax import lax
from jax.experimental import pallas as pl
from jax.experimental.pallas import tpu as pltpu
from jax.experimental.pallas import tpu_sc as plsc

N_NODES = 10000
D_FEAT = 128
N_EDGES = 320000

NC = 2
NS = 16
NW = NC * NS
CHUNK = 128
K = 80
E_PAD = NW * K * CHUNK
PAD = E_PAD - N_EDGES
N_PAD = 10240
ROWS_PER_TILE = N_PAD // NS
DUMMY = 10016


GROUP = 8
K0 = 160
NP = K0 // (2 * GROUP)


def _sc_pass_body(src_hbm, dst_hbm, table_hbm,
                  sum_out, cnt_out,
                  is0, is1, id0, id1, rows0, rows1, counts_v, acc,
                  gs0, gs1, ss0, ss1, sis0, sis1, sid0, sid1):
    isr = (is0, is1)
    idr = (id0, id1)
    rows = (rows0, rows1)
    gsem = (gs0, gs1)
    ssem = (ss0, ss1)
    isem = (sis0, sis1)
    dsem = (sid0, sid1)
    c = lax.axis_index("c")
    s = lax.axis_index("s")
    wid = c * NS + s
    zv = jnp.zeros((16,), jnp.float32)
    ones = jnp.ones((16,), jnp.float32)

    def zrow(i, carry):
        for r in range(8):
            rows0[i, pl.ds(r * 16, 16)] = zv
        return carry

    lax.fori_loop(0, CHUNK, zrow, None)

    def zcnt(i, carry):
        counts_v[pl.ds(i * 16, 16)] = zv
        return carry

    lax.fori_loop(0, N_PAD // 16, zcnt, None)
    for t in range(ROWS_PER_TILE // CHUNK):
        pltpu.sync_copy(rows0, acc.at[pl.ds(s * ROWS_PER_TILE + t * CHUNK, CHUNK)])
    plsc.subcore_barrier()

    is_c0 = c == 0
    base = s * K0

    def start_idx(g, p):
        pltpu.async_copy(src_hbm.at[pl.ds(base + g * GROUP, GROUP)], isr[p], isem[p])
        pltpu.async_copy(dst_hbm.at[pl.ds(base + g * GROUP, GROUP)], idr[p], dsem[p])

    def wait_idx(p):
        pltpu.make_async_copy(src_hbm.at[pl.ds(base, GROUP)], isr[p], isem[p]).wait()
        pltpu.make_async_copy(dst_hbm.at[pl.ds(base, GROUP)], idr[p], dsem[p]).wait()

    def start_gather(p, t, b):
        pltpu.async_copy(table_hbm.at[isr[p].at[t]], rows[b], gsem[b])

    def wait_gather(p, t, b):
        pltpu.make_async_copy(table_hbm.at[isr[p].at[t]], rows[b], gsem[b]).wait()

    def start_scatter(p, t, b):
        pltpu.async_copy(rows[b], acc.at[idr[p].at[t]], ssem[b], add=True)

    def wait_scatter(p, t, b):
        pltpu.make_async_copy(rows[b], acc.at[idr[p].at[t]], ssem[b]).wait()

    def counts(p, t):
        for r in range(8):
            v = idr[p][t, pl.ds(r * 16, 16)]
            plsc.addupdate_scatter(counts_v, [v], ones)

    def group_step(p, g_next, first=False, last=False):
        for t in range(GROUP):
            b = t % 2
            wait_gather(p, t, b)
            start_scatter(p, t, b)
            counts(p, t)
            if not (first and t == 0):
                wait_scatter(p, t, 1 - b)
            if t == 0 and g_next is not None:
                start_idx(g_next, 1 - p)
            if t == GROUP - 1:
                if g_next is not None:
                    wait_idx(1 - p)
                    start_gather(1 - p, 0, 1 - b)
            else:
                start_gather(p, t + 1, 1 - b)
        if last:
            wait_scatter(p, GROUP - 1, 1)

    @pl.when(is_c0)
    def _pipeline():
        start_idx(0, 0)
        wait_idx(0)
        start_gather(0, 0, 0)
        group_step(0, 1, first=True)
        group_step(1, 2)

        def pair(gg, carry):
            group_step(0, 2 * gg + 1)
            group_step(1, 2 * gg + 2)
            return carry

        lax.fori_loop(1, NP - 1, pair, None)

        group_step(0, 2 * NP - 1)
        group_step(1, None, last=True)

    plsc.subcore_barrier()

    sl = pl.ds(s * ROWS_PER_TILE, ROWS_PER_TILE)
    pltpu.sync_copy(acc.at[sl], sum_out.at[c].at[sl])
    pltpu.sync_copy(counts_v, cnt_out.at[wid])


def _sc_pass(src2d, dst2d, table):
    mesh = plsc.VectorSubcoreMesh(core_axis_name="c", subcore_axis_name="s")
    return pl.kernel(
        _sc_pass_body,
        out_type=[
            jax.ShapeDtypeStruct((NC, N_PAD, D_FEAT), jnp.float32),
            jax.ShapeDtypeStruct((NW, N_PAD), jnp.float32),
        ],
        mesh=mesh,
        compiler_params=pltpu.CompilerParams(needs_layout_passes=False),
        scratch_types=(
            [pltpu.VMEM((GROUP, CHUNK), jnp.int32)] * 4
            + [pltpu.VMEM((CHUNK, D_FEAT), jnp.float32)] * 2
            + [
                pltpu.VMEM((N_PAD,), jnp.float32),
                pltpu.VMEM_SHARED((N_PAD, D_FEAT), jnp.float32),
            ]
            + [pltpu.SemaphoreType.DMA] * 8
        ),
    )(src2d, dst2d, table)


def _dense_body(sp_ref, cp_ref, l_ref, dstm_ref, t1_ref):
    summed = sp_ref[0] + sp_ref[1]
    counts = jnp.sum(cp_ref[...], axis=0)
    avg = summed / jnp.clip(counts, 1.0, None)[:, None]
    p = avg * avg
    rs = jnp.sum(p, axis=1, keepdims=True)
    sharp = p / jnp.maximum(rs, 1e-30)
    row = lax.broadcasted_iota(jnp.int32, (N_PAD, 1), 0)
    dstm = jnp.where(row < N_NODES, sharp + 1e-10, 0.0)
    l_ref[...] = jnp.log(avg + 1e-10)
    dstm_ref[...] = dstm
    a = jnp.sum(dstm * jnp.log(jnp.maximum(dstm, 1e-30)), axis=1)
    t1_ref[...] = jnp.sum(counts * a).reshape(1, 1)


def _dense(sum_p, cnt_p):
    return pl.pallas_call(
        _dense_body,
        out_shape=[
            jax.ShapeDtypeStruct((N_PAD, D_FEAT), jnp.float32),
            jax.ShapeDtypeStruct((N_PAD, D_FEAT), jnp.float32),
            jax.ShapeDtypeStruct((1, 1), jnp.float32),
        ],
    )(sum_p, cnt_p)


def _final_body(sp2_ref, dstm_ref, t1_ref, out_ref):
    s = sp2_ref[0] + sp2_ref[1]
    term2 = jnp.sum(s * dstm_ref[...])
    out_ref[...] = ((t1_ref[0, 0] - term2) / float(N_EDGES)).reshape(1, 1)


def _final(s_p, dstm, t1):
    return pl.pallas_call(
        _final_body,
        out_shape=jax.ShapeDtypeStruct((1, 1), jnp.float32),
    )(s_p, dstm, t1)


def kernel(edge_index, aug_pred):
    src = edge_index[0].astype(jnp.int32)
    dst = edge_index[1].astype(jnp.int32)
    src2d = jnp.concatenate(
        [src, jnp.zeros((PAD,), jnp.int32)]).reshape(NW * K, CHUNK)
    dst2d = jnp.concatenate(
        [dst, jnp.full((PAD,), DUMMY, jnp.int32)]).reshape(NW * K, CHUNK)
    sum_p, cnt_p = _sc_pass(src2d, dst2d, aug_pred)
    l_tab, dstm, t1 = _dense(sum_p, cnt_p)
    s_p, _ = _sc_pass(src2d, dst2d, l_tab[:N_NODES])
    loss = _final(s_p, dstm, t1)
    return loss.reshape(())


# speedup vs baseline: 1.4240x; 1.4240x over previous
import functools

import jax
import jax.numpy as jnp
from jax import lax
from jax.experimental import pallas as pl
from jax.experimental.pallas import tpu as pltpu
from jax.experimental.pallas import tpu_sc as plsc

N_NODES = 10000
D_FEAT = 128
N_EDGES = 320000

NC = 2
NS = 16
NW = NC * NS
CHUNK = 128
K = 80
E_PAD = NW * K * CHUNK
PAD = E_PAD - N_EDGES
N_PAD = 10240
ROWS_PER_TILE = N_PAD // NS
DUMMY = 10016


GROUP = 8
K0 = 144
K1 = 16
NCH0 = NS * K0
NP0 = K0 // (2 * GROUP)


def _sc_pass_body(src_hbm, dst_hbm, table_hbm,
                  sum_out, cnt_out,
                  is0, is1, id0, id1, rows0, rows1, counts_v, acc,
                  gs0, gs1, ss0, ss1, sis0, sis1, sid0, sid1):
    isr = (is0, is1)
    idr = (id0, id1)
    rows = (rows0, rows1)
    gsem = (gs0, gs1)
    ssem = (ss0, ss1)
    isem = (sis0, sis1)
    dsem = (sid0, sid1)
    c = lax.axis_index("c")
    s = lax.axis_index("s")
    wid = c * NS + s
    zv = jnp.zeros((16,), jnp.float32)
    ones = jnp.ones((16,), jnp.float32)

    def zrow(i, carry):
        for r in range(8):
            rows0[i, pl.ds(r * 16, 16)] = zv
        return carry

    lax.fori_loop(0, CHUNK, zrow, None)

    def zcnt(i, carry):
        counts_v[pl.ds(i * 16, 16)] = zv
        return carry

    lax.fori_loop(0, N_PAD // 16, zcnt, None)
    for t in range(ROWS_PER_TILE // CHUNK):
        pltpu.sync_copy(rows0, acc.at[pl.ds(s * ROWS_PER_TILE + t * CHUNK, CHUNK)])
    plsc.subcore_barrier()

    is_c0 = c == 0
    base0 = s * K0
    base1 = NCH0 + s * K1

    def start_idx(base, g, p):
        pltpu.async_copy(src_hbm.at[pl.ds(base + g * GROUP, GROUP)], isr[p], isem[p])
        pltpu.async_copy(dst_hbm.at[pl.ds(base + g * GROUP, GROUP)], idr[p], dsem[p])

    def wait_idx(p):
        pltpu.make_async_copy(src_hbm.at[pl.ds(0, GROUP)], isr[p], isem[p]).wait()
        pltpu.make_async_copy(dst_hbm.at[pl.ds(0, GROUP)], idr[p], dsem[p]).wait()

    def start_gather(p, t, b):
        pltpu.async_copy(table_hbm.at[isr[p].at[t]], rows[b], gsem[b])

    def wait_gather(p, t, b):
        pltpu.make_async_copy(table_hbm.at[isr[p].at[t]], rows[b], gsem[b]).wait()

    def start_scatter(p, t, b):
        pltpu.async_copy(rows[b], acc.at[idr[p].at[t]], ssem[b], add=True)

    def wait_scatter(p, t, b):
        pltpu.make_async_copy(rows[b], acc.at[idr[p].at[t]], ssem[b]).wait()

    def counts(p, t):
        for r in range(8):
            v = idr[p][t, pl.ds(r * 16, 16)]
            plsc.addupdate_scatter(counts_v, [v], ones)

    def group_step(base, p, g_next, first=False, last=False):
        for t in range(GROUP):
            b = t % 2
            wait_gather(p, t, b)
            start_scatter(p, t, b)
            counts(p, t)
            if not (first and t == 0):
                wait_scatter(p, t, 1 - b)
            if t == 0 and g_next is not None:
                start_idx(base, g_next, 1 - p)
            if t == GROUP - 1:
                if g_next is not None:
                    wait_idx(1 - p)
                    start_gather(1 - p, 0, 1 - b)
            else:
                start_gather(p, t + 1, 1 - b)
        if last:
            wait_scatter(p, GROUP - 1, 1)

    @pl.when(is_c0)
    def _pipeline0():
        start_idx(base0, 0, 0)
        wait_idx(0)
        start_gather(0, 0, 0)
        group_step(base0, 0, 1, first=True)
        group_step(base0, 1, 2)

        def pair(gg, carry):
            group_step(base0, 0, 2 * gg + 1)
            group_step(base0, 1, 2 * gg + 2)
            return carry

        lax.fori_loop(1, NP0 - 1, pair, None)

        group_step(base0, 0, 2 * NP0 - 1)
        group_step(base0, 1, None, last=True)

    @pl.when(jnp.logical_not(is_c0))
    def _pipeline1():
        start_idx(base1, 0, 0)
        wait_idx(0)
        start_gather(0, 0, 0)
        group_step(base1, 0, 1, first=True)
        group_step(base1, 1, None, last=True)

    plsc.subcore_barrier()

    sl = pl.ds(s * ROWS_PER_TILE, ROWS_PER_TILE)
    pltpu.sync_copy(acc.at[sl], sum_out.at[c].at[sl])
    pltpu.sync_copy(counts_v, cnt_out.at[wid])


def _sc_pass(src2d, dst2d, table):
    mesh = plsc.VectorSubcoreMesh(core_axis_name="c", subcore_axis_name="s")
    return pl.kernel(
        _sc_pass_body,
        out_type=[
            jax.ShapeDtypeStruct((NC, N_PAD, D_FEAT), jnp.float32),
            jax.ShapeDtypeStruct((NW, N_PAD), jnp.float32),
        ],
        mesh=mesh,
        compiler_params=pltpu.CompilerParams(needs_layout_passes=False),
        scratch_types=(
            [pltpu.VMEM((GROUP, CHUNK), jnp.int32)] * 4
            + [pltpu.VMEM((CHUNK, D_FEAT), jnp.float32)] * 2
            + [
                pltpu.VMEM((N_PAD,), jnp.float32),
                pltpu.VMEM_SHARED((N_PAD, D_FEAT), jnp.float32),
            ]
            + [pltpu.SemaphoreType.DMA] * 8
        ),
    )(src2d, dst2d, table)


def _dense_body(sp_ref, cp_ref, l_ref, dstm_ref, t1_ref):
    summed = sp_ref[0] + sp_ref[1]
    counts = jnp.sum(cp_ref[...], axis=0)
    avg = summed / jnp.clip(counts, 1.0, None)[:, None]
    p = avg * avg
    rs = jnp.sum(p, axis=1, keepdims=True)
    sharp = p / jnp.maximum(rs, 1e-30)
    row = lax.broadcasted_iota(jnp.int32, (N_PAD, 1), 0)
    dstm = jnp.where(row < N_NODES, sharp + 1e-10, 0.0)
    l_ref[...] = jnp.log(avg + 1e-10)
    dstm_ref[...] = dstm
    a = jnp.sum(dstm * jnp.log(jnp.maximum(dstm, 1e-30)), axis=1)
    t1_ref[...] = jnp.sum(counts * a).reshape(1, 1)


def _dense(sum_p, cnt_p):
    return pl.pallas_call(
        _dense_body,
        out_shape=[
            jax.ShapeDtypeStruct((N_PAD, D_FEAT), jnp.float32),
            jax.ShapeDtypeStruct((N_PAD, D_FEAT), jnp.float32),
            jax.ShapeDtypeStruct((1, 1), jnp.float32),
        ],
    )(sum_p, cnt_p)


def _final_body(sp2_ref, dstm_ref, t1_ref, out_ref):
    s = sp2_ref[0] + sp2_ref[1]
    term2 = jnp.sum(s * dstm_ref[...])
    out_ref[...] = ((t1_ref[0, 0] - term2) / float(N_EDGES)).reshape(1, 1)


def _final(s_p, dstm, t1):
    return pl.pallas_call(
        _final_body,
        out_shape=jax.ShapeDtypeStruct((1, 1), jnp.float32),
    )(s_p, dstm, t1)


def kernel(edge_index, aug_pred):
    src = edge_index[0].astype(jnp.int32)
    dst = edge_index[1].astype(jnp.int32)
    src2d = jnp.concatenate(
        [src, jnp.zeros((PAD,), jnp.int32)]).reshape(NW * K, CHUNK)
    dst2d = jnp.concatenate(
        [dst, jnp.full((PAD,), DUMMY, jnp.int32)]).reshape(NW * K, CHUNK)
    sum_p, cnt_p = _sc_pass(src2d, dst2d, aug_pred)
    l_tab, dstm, t1 = _dense(sum_p, cnt_p)
    s_p, _ = _sc_pass(src2d, dst2d, l_tab[:N_NODES])
    loss = _final(s_p, dstm, t1)
    return loss.reshape(())
